# baseline (speedup 1.0000x reference)
import functools

import jax
import jax.numpy as jnp
from jax import lax
from jax.experimental import pallas as pl
from jax.experimental.pallas import tpu as pltpu
from jax.experimental.pallas import tpu_sc as plsc

_NC = 2
_NS = 16
_CH = 128


def kernel(user_idx, year, num_ratings, user_table, year_table,
           rating_table, year_bounds, rating_bounds):
    B = user_idx.shape[0]
    E = user_table.shape[1]
    C = 3 * E
    nbnd = year_bounds.shape[0]
    nbins = year_table.shape[0]
    NW = _NC * _NS
    bpw = B // NW
    nch = bpw // _CH
    mesh = plsc.VectorSubcoreMesh(core_axis_name="c", subcore_axis_name="s")

    @functools.partial(
        pl.kernel,
        mesh=mesh,
        out_type=jax.ShapeDtypeStruct((C, B), jnp.float32),
        compiler_params=pltpu.CompilerParams(use_tc_tiling_on_sc=False),
        scratch_types=[
            pltpu.VMEM((nch, _CH), jnp.int32),
            pltpu.VMEM((nch, _CH), jnp.float32),
            pltpu.VMEM((nch, _CH), jnp.float32),
            pltpu.VMEM((32,), jnp.float32),
            pltpu.VMEM((32,), jnp.float32),
            pltpu.VMEM((nch, C, _CH), jnp.int32),
            pltpu.VMEM((nch, C, _CH), jnp.float32),
            pltpu.SemaphoreType.DMA,
        ],
    )
    def sc_kernel(uidx_h, year_h, rate_h, utab_h, ytab_h, rtab_h,
                  ybnd_h, rbnd_h, out_h,
                  idx_v, yv_v, rv_v, ybnd_v, rbnd_v, gidx_v, colblk, sem):
        wid = lax.axis_index("s") * _NC + lax.axis_index("c")
        base = wid * bpw

        for j in range(nch):
            pltpu.sync_copy(uidx_h.at[pl.ds(base + j * _CH, _CH)], idx_v.at[j])
            pltpu.sync_copy(year_h.at[pl.ds(base + j * _CH, _CH)], yv_v.at[j])
            pltpu.sync_copy(rate_h.at[pl.ds(base + j * _CH, _CH)], rv_v.at[j])
        pltpu.sync_copy(ybnd_h, ybnd_v.at[pl.ds(0, nbnd)])
        pltpu.sync_copy(rbnd_h, rbnd_v.at[pl.ds(0, nbnd)])

        lane = lax.iota(jnp.int32, 16)
        yb0 = ybnd_v[pl.ds(0, 16)]
        yb1 = ybnd_v[pl.ds(16, 16)]
        rb0 = rbnd_v[pl.ds(0, 16)]
        rb1 = rbnd_v[pl.ds(16, 16)]

        gdn = lax.GatherDimensionNumbers(
            offset_dims=(), collapsed_slice_dims=(0,), start_index_map=(0,))

        def bcast(vec, idx):
            return lax.gather(vec, idx.reshape(16, 1), gdn, (1,),
                              mode=lax.GatherScatterMode.PROMISE_IN_BOUNDS)

        def rank(b0, b1, v):
            pos = jnp.zeros((16,), jnp.int32)
            for sz in (16, 8, 4, 2, 1):
                nxt = pos + sz
                probe = jnp.minimum(nxt - 1, nbnd - 1)
                g0 = bcast(b0, jnp.minimum(probe, 15))
                g1 = bcast(b1, jnp.clip(probe - 16, 0, 15))
                bv = jnp.where(probe < 16, g0, g1)
                take = (nxt <= nbnd) & (bv <= v)
                pos = jnp.where(take, nxt, pos)
            return jnp.minimum(pos, nbins - 1)

        cps = []
        for j in range(nch):
            def cbody(i8, carry, j=j):
                off = i8 * 16
                u10 = idx_v[j, pl.ds(off, 16)] * E
                y10 = rank(yb0, yb1, yv_v[j, pl.ds(off, 16)]) * E
                r10 = rank(rb0, rb1, rv_v[j, pl.ds(off, 16)]) * E
                for c in range(E):
                    gidx_v[j, c, pl.ds(off, 16)] = u10 + c
                    gidx_v[j, E + c, pl.ds(off, 16)] = y10 + c
                    gidx_v[j, 2 * E + c, pl.ds(off, 16)] = r10 + c
                return carry
            lax.fori_loop(0, _CH // 16, cbody, 0)
            for c in range(E):
                cps.append(pltpu.async_copy(
                    utab_h.at[gidx_v.at[j, c]], colblk.at[j, c], sem))
                cps.append(pltpu.async_copy(
                    ytab_h.at[gidx_v.at[j, E + c]], colblk.at[j, E + c], sem))
                cps.append(pltpu.async_copy(
                    rtab_h.at[gidx_v.at[j, 2 * E + c]], colblk.at[j, 2 * E + c],
                    sem))
        for c in cps:
            c.wait()

        for j in range(nch):
            pltpu.sync_copy(
                colblk.at[j],
                out_h.at[pl.ds(0, C), pl.ds(base + j * _CH, _CH)])

    out = sc_kernel(user_idx, year, num_ratings, user_table.reshape(-1),
                    year_table.reshape(-1), rating_table.reshape(-1),
                    year_bounds, rating_bounds)
    return out.T
